# baseline (speedup 1.0000x reference)
import functools

import jax
import jax.numpy as jnp
import numpy as np
from jax import lax
from jax.experimental import pallas as pl
from jax.experimental.pallas import tpu as pltpu
from jax.experimental.pallas import tpu_sc as plsc

_CAT_DIMS = [100000] * 26
_NF = 26
_BATCH = 4096
_ED = 16
_OD = 8
_ROWS = _BATCH * _NF
_CHUNK = 128
_NW = 32
_CPW = _ROWS // _CHUNK // _NW
_TC = _ROWS // _CHUNK

_OFFSETS = np.cumsum([0] + _CAT_DIMS[:-1]).astype(np.int32)


def _sc_gather(idx_hbm, et, lt, eg4, lg4, idx_v, ebuf, lbuf, sem_e, sem_l):
  wid = lax.axis_index("s") * 2 + lax.axis_index("c")
  base = wid * _CPW
  pltpu.sync_copy(idx_hbm.at[wid], idx_v)

  def _drain_plane(sem):
    pltpu.make_async_copy(idx_hbm.at[wid], idx_v, sem).wait()

  def _fire_embed(e):
    tr, ep = e // 8, e % 8

    def body(j, carry):
      pltpu.async_copy(et.at[e].at[idx_v.at[j]], ebuf.at[tr].at[j].at[ep],
                       sem_e)
      return carry

    lax.fori_loop(0, _CPW, body, 0)

  def _fire_lin(e):
    def body(j, carry):
      pltpu.async_copy(lt.at[e].at[idx_v.at[j]], lbuf.at[j].at[e], sem_l)
      return carry

    lax.fori_loop(0, _CPW, body, 0)

  _fire_embed(0)
  for e in range(1, _ED):
    _fire_embed(e)
    _drain_plane(sem_e)
  _fire_lin(0)
  for e in range(1, _OD):
    _fire_lin(e)
    _drain_plane(sem_l)
  _drain_plane(sem_e)
  _drain_plane(sem_l)

  pltpu.sync_copy(ebuf.at[0], eg4.at[0].at[pl.ds(base, _CPW)])
  pltpu.sync_copy(ebuf.at[1], eg4.at[1].at[pl.ds(base, _CPW)])
  pltpu.sync_copy(lbuf, lg4.at[pl.ds(base, _CPW)])


@functools.lru_cache(maxsize=1)
def _sc_gather_call():
  return pl.kernel(
      _sc_gather,
      out_type=(
          jax.ShapeDtypeStruct((2, _TC, 8, _CHUNK), jnp.float32),
          jax.ShapeDtypeStruct((_TC, 8, _CHUNK), jnp.float32),
      ),
      mesh=plsc.VectorSubcoreMesh(core_axis_name="c", subcore_axis_name="s"),
      scratch_types=[
          pltpu.VMEM((_CPW, _CHUNK), jnp.int32),
          pltpu.VMEM((2, _CPW, 8, _CHUNK), jnp.float32),
          pltpu.VMEM((_CPW, 8, _CHUNK), jnp.float32),
          pltpu.SemaphoreType.DMA,
          pltpu.SemaphoreType.DMA,
      ],
      compiler_params=pltpu.CompilerParams(use_tc_tiling_on_sc=False),
  )


_NBLK = 8
_BN = _BATCH // _NBLK


def _tc_body(w_ref, e_ref, l_ref, out_ref, sacc, qacc, lacc):
  c = pl.program_id(1)
  eb = e_ref[...]
  lb = l_ref[...]
  ebsq = jnp.sum(eb * eb, axis=0)

  @pl.when(c == 0)
  def _():
    for o in range(_OD):
      w = w_ref[o, 0]
      sacc[o, :, :] = w * eb
      qacc[o, :] = (w * w) * ebsq
    lacc[...] = lb

  @pl.when(c != 0)
  def _():
    for o in range(_OD):
      w = w_ref[o, c]
      sacc[o, :, :] = sacc[o, :, :] + w * eb
      qacc[o, :] = qacc[o, :] + (w * w) * ebsq
    lacc[...] = lacc[...] + lb

  @pl.when(c == _NF - 1)
  def _():
    for o in range(_OD):
      s = sacc[o, :, :]
      out_ref[o, :] = lacc[o, :] + 0.5 * (jnp.sum(s * s, axis=0) - qacc[o, :])


def kernel(cat, lin_table, embed_table, project_weight):
  cat = jnp.asarray(cat, jnp.int32)
  idx = cat.T + jnp.asarray(_OFFSETS)[:, None]
  idx3 = idx.reshape(_NW, _CPW, _CHUNK)

  et = embed_table.T
  lt = lin_table.T
  eg4, lg4 = _sc_gather_call()(idx3, et, lt)
  eg = eg4.transpose(0, 2, 1, 3).reshape(_ED, _ROWS)
  lg = lg4.transpose(1, 0, 2).reshape(_OD, _ROWS)

  out = pl.pallas_call(
      _tc_body,
      grid=(_NBLK, _NF),
      in_specs=[
          pl.BlockSpec(memory_space=pltpu.SMEM),
          pl.BlockSpec((_ED, _BN), lambda j, c: (0, c * (_BATCH // _BN) + j)),
          pl.BlockSpec((_OD, _BN), lambda j, c: (0, c * (_BATCH // _BN) + j)),
      ],
      out_specs=pl.BlockSpec((_OD, _BN), lambda j, c: (0, j)),
      out_shape=jax.ShapeDtypeStruct((_OD, _BATCH), jnp.float32),
      scratch_shapes=[
          pltpu.VMEM((_OD, _ED, _BN), jnp.float32),
          pltpu.VMEM((_OD, _BN), jnp.float32),
          pltpu.VMEM((_OD, _BN), jnp.float32),
      ],
  )(project_weight, eg, lg)
  return out.T


# speedup vs baseline: 7.7113x; 7.7113x over previous
import functools

import jax
import jax.numpy as jnp
import numpy as np
from jax import lax
from jax.experimental import pallas as pl
from jax.experimental.pallas import tpu as pltpu
from jax.experimental.pallas import tpu_sc as plsc

_CAT_DIMS = [100000] * 26
_NF = 26
_BATCH = 4096
_ED = 16
_OD = 8
_ROWS = _BATCH * _NF
_CHUNK = 128
_NW = 32
_CPW = _ROWS // _CHUNK // _NW
_TC = _ROWS // _CHUNK

_OFFSETS = np.cumsum([0] + _CAT_DIMS[:-1]).astype(np.int32)


_CB = 131072
_NCB = -(-2600000 // _CB)
_PSTRIDE = _NCB * _CB


def _detile_body(in_ref, out_ref):
  p = pl.program_id(1)
  eb = in_ref[...]
  for q in range(8):
    @pl.when(p % 8 == q)
    def _():
      out_ref[...] = eb[q, :]


def _detile(tab, nplanes):
  return pl.pallas_call(
      _detile_body,
      grid=(_NCB, nplanes),
      in_specs=[pl.BlockSpec((8, _CB), lambda i, p: (p // 8, i))],
      out_specs=pl.BlockSpec((_CB,), lambda i, p: (p * _NCB + i,)),
      out_shape=jax.ShapeDtypeStruct((nplanes * _PSTRIDE,), jnp.float32),
  )(tab)


def _sc_gather(idx_hbm, et, lt, eg4, lg4, idx_v, ebuf, lbuf, sem_e, sem_l):
  wid = lax.axis_index("s") * 2 + lax.axis_index("c")
  base = wid * _CPW
  pltpu.sync_copy(idx_hbm.at[wid], idx_v)

  def _drain_plane(sem):
    pltpu.make_async_copy(idx_hbm.at[wid], idx_v, sem).wait()

  def _fire_embed(e):
    tr, ep = e // 8, e % 8

    def body(j, carry):
      pltpu.async_copy(et.at[e].at[idx_v.at[j]], ebuf.at[tr].at[j].at[ep],
                       sem_e)
      return carry

    lax.fori_loop(0, _CPW, body, 0)

  def _fire_lin(e):
    def body(j, carry):
      pltpu.async_copy(lt.at[e].at[idx_v.at[j]], lbuf.at[j].at[e], sem_l)
      return carry

    lax.fori_loop(0, _CPW, body, 0)

  _fire_embed(0)
  for e in range(1, _ED):
    _fire_embed(e)
    _drain_plane(sem_e)
  _fire_lin(0)
  for e in range(1, _OD):
    _fire_lin(e)
    _drain_plane(sem_l)
  _drain_plane(sem_e)
  _drain_plane(sem_l)

  pltpu.sync_copy(ebuf.at[0], eg4.at[0].at[pl.ds(base, _CPW)])
  pltpu.sync_copy(ebuf.at[1], eg4.at[1].at[pl.ds(base, _CPW)])
  pltpu.sync_copy(lbuf, lg4.at[pl.ds(base, _CPW)])


@functools.lru_cache(maxsize=1)
def _sc_gather_call():
  return pl.kernel(
      _sc_gather,
      out_type=(
          jax.ShapeDtypeStruct((2, _TC, 8, _CHUNK), jnp.float32),
          jax.ShapeDtypeStruct((_TC, 8, _CHUNK), jnp.float32),
      ),
      mesh=plsc.VectorSubcoreMesh(core_axis_name="c", subcore_axis_name="s"),
      scratch_types=[
          pltpu.VMEM((_CPW, _CHUNK), jnp.int32),
          pltpu.VMEM((2, _CPW, 8, _CHUNK), jnp.float32),
          pltpu.VMEM((_CPW, 8, _CHUNK), jnp.float32),
          pltpu.SemaphoreType.DMA,
          pltpu.SemaphoreType.DMA,
      ],
      compiler_params=pltpu.CompilerParams(use_tc_tiling_on_sc=False),
  )


_NBLK = 8
_BN = _BATCH // _NBLK


def _tc_body(w_ref, e_ref, l_ref, out_ref, sacc, qacc, lacc):
  c = pl.program_id(1)
  eb = e_ref[...]
  lb = l_ref[...]
  ebsq = jnp.sum(eb * eb, axis=0)

  @pl.when(c == 0)
  def _():
    for o in range(_OD):
      w = w_ref[o, 0]
      sacc[o, :, :] = w * eb
      qacc[o, :] = (w * w) * ebsq
    lacc[...] = lb

  @pl.when(c != 0)
  def _():
    for o in range(_OD):
      w = w_ref[o, c]
      sacc[o, :, :] = sacc[o, :, :] + w * eb
      qacc[o, :] = qacc[o, :] + (w * w) * ebsq
    lacc[...] = lacc[...] + lb

  @pl.when(c == _NF - 1)
  def _():
    for o in range(_OD):
      s = sacc[o, :, :]
      out_ref[o, :] = lacc[o, :] + 0.5 * (jnp.sum(s * s, axis=0) - qacc[o, :])


def kernel(cat, lin_table, embed_table, project_weight):
  cat = jnp.asarray(cat, jnp.int32)
  idx = cat.T + jnp.asarray(_OFFSETS)[:, None]
  idx3 = idx.reshape(_NW, _CPW, _CHUNK)

  et = embed_table.T
  lt = lin_table.T
  etl = _detile(et, _ED).reshape(_ED, _PSTRIDE)
  ltl = _detile(lt, _OD).reshape(_OD, _PSTRIDE)
  eg4, lg4 = _sc_gather_call()(idx3, etl, ltl)
  eg = eg4.transpose(0, 2, 1, 3).reshape(_ED, _ROWS)
  lg = lg4.transpose(1, 0, 2).reshape(_OD, _ROWS)

  out = pl.pallas_call(
      _tc_body,
      grid=(_NBLK, _NF),
      in_specs=[
          pl.BlockSpec(memory_space=pltpu.SMEM),
          pl.BlockSpec((_ED, _BN), lambda j, c: (0, c * (_BATCH // _BN) + j)),
          pl.BlockSpec((_OD, _BN), lambda j, c: (0, c * (_BATCH // _BN) + j)),
      ],
      out_specs=pl.BlockSpec((_OD, _BN), lambda j, c: (0, j)),
      out_shape=jax.ShapeDtypeStruct((_OD, _BATCH), jnp.float32),
      scratch_shapes=[
          pltpu.VMEM((_OD, _ED, _BN), jnp.float32),
          pltpu.VMEM((_OD, _BN), jnp.float32),
          pltpu.VMEM((_OD, _BN), jnp.float32),
      ],
  )(project_weight, eg, lg)
  return out.T


# speedup vs baseline: 8.5324x; 1.1065x over previous
import functools

import jax
import jax.numpy as jnp
import numpy as np
from jax import lax
from jax.experimental import pallas as pl
from jax.experimental.pallas import tpu as pltpu
from jax.experimental.pallas import tpu_sc as plsc

_CAT_DIMS = [100000] * 26
_NF = 26
_BATCH = 4096
_ED = 16
_OD = 8
_ROWS = _BATCH * _NF
_CHUNK = 128
_NW = 32
_CPW = _ROWS // _CHUNK // _NW
_TCOL = _ROWS // _CHUNK

_OFFSETS = np.cumsum([0] + _CAT_DIMS[:-1]).astype(np.int32)

_CB = 131072
_NCB = -(-2600000 // _CB)
_PSTRIDE = _NCB * _CB


def _detile_body(in_ref, out_ref):
  p = pl.program_id(1)
  eb = in_ref[...]
  for q in range(8):
    @pl.when(p % 8 == q)
    def _():
      out_ref[...] = eb[q, :]


def _detile(tab, trow):
  return pl.pallas_call(
      _detile_body,
      grid=(_NCB, 8),
      in_specs=[pl.BlockSpec((8, _CB), lambda i, p, trow=trow: (trow, i))],
      out_specs=pl.BlockSpec((_CB,), lambda i, p: (p * _NCB + i,)),
      out_shape=jax.ShapeDtypeStruct((8 * _PSTRIDE,), jnp.float32),
  )(tab)


def _sc_gather(idx_hbm, tab, g4, idx_v, buf, sem):
  wid = lax.axis_index("s") * 2 + lax.axis_index("c")
  base = wid * _CPW
  pltpu.sync_copy(idx_hbm.at[wid], idx_v)

  def _drain_plane():
    pltpu.make_async_copy(idx_hbm.at[wid], idx_v, sem).wait()

  def _fire(e):
    def body(j, carry):
      pltpu.async_copy(tab.at[e].at[idx_v.at[j]], buf.at[j].at[e], sem)
      return carry

    lax.fori_loop(0, _CPW, body, 0)

  _fire(0)
  for e in range(1, _OD):
    _fire(e)
    _drain_plane()
  _drain_plane()
  pltpu.sync_copy(buf, g4.at[pl.ds(base, _CPW)])


@functools.lru_cache(maxsize=1)
def _sc_gather_call():
  return pl.kernel(
      _sc_gather,
      out_type=jax.ShapeDtypeStruct((_TCOL, _OD, _CHUNK), jnp.float32),
      mesh=plsc.VectorSubcoreMesh(core_axis_name="c", subcore_axis_name="s"),
      scratch_types=[
          pltpu.VMEM((_CPW, _CHUNK), jnp.int32),
          pltpu.VMEM((_CPW, _OD, _CHUNK), jnp.float32),
          pltpu.SemaphoreType.DMA,
      ],
      compiler_params=pltpu.CompilerParams(use_tc_tiling_on_sc=False),
  )


_NBLK = 8
_BN = _BATCH // _NBLK


def _tc_body(w_ref, elo_ref, ehi_ref, l_ref, out_ref, sacc, qacc, lacc):
  c = pl.program_id(1)
  lo = elo_ref[...]
  hi = ehi_ref[...]
  lb = l_ref[...]
  ebsq = jnp.sum(lo * lo, axis=0) + jnp.sum(hi * hi, axis=0)

  @pl.when(c == 0)
  def _():
    for o in range(_OD):
      w = w_ref[o, 0]
      sacc[o, 0:8, :] = w * lo
      sacc[o, 8:16, :] = w * hi
      qacc[o, :] = (w * w) * ebsq
    lacc[...] = lb

  @pl.when(c != 0)
  def _():
    for o in range(_OD):
      w = w_ref[o, c]
      sacc[o, 0:8, :] = sacc[o, 0:8, :] + w * lo
      sacc[o, 8:16, :] = sacc[o, 8:16, :] + w * hi
      qacc[o, :] = qacc[o, :] + (w * w) * ebsq
    lacc[...] = lacc[...] + lb

  @pl.when(c == _NF - 1)
  def _():
    for o in range(_OD):
      s = sacc[o, :, :]
      out_ref[o, :] = lacc[o, :] + 0.5 * (jnp.sum(s * s, axis=0) - qacc[o, :])


def kernel(cat, lin_table, embed_table, project_weight):
  cat = jnp.asarray(cat, jnp.int32)
  idx = cat.T + jnp.asarray(_OFFSETS)[:, None]
  idx3 = idx.reshape(_NW, _CPW, _CHUNK)

  et = embed_table.T
  lt = lin_table.T
  ltl = _detile(lt, 0).reshape(_OD, _PSTRIDE)
  elo = _detile(et, 0).reshape(_OD, _PSTRIDE)
  ehi = _detile(et, 1).reshape(_OD, _PSTRIDE)

  gather = _sc_gather_call()
  lg4 = gather(idx3, ltl)
  eg4_lo = gather(idx3, elo)
  eg4_hi = gather(idx3, ehi)
  lg = lg4.transpose(1, 0, 2).reshape(_OD, _ROWS)
  eg_lo = eg4_lo.transpose(1, 0, 2).reshape(_OD, _ROWS)
  eg_hi = eg4_hi.transpose(1, 0, 2).reshape(_OD, _ROWS)

  out = pl.pallas_call(
      _tc_body,
      grid=(_NBLK, _NF),
      in_specs=[
          pl.BlockSpec(memory_space=pltpu.SMEM),
          pl.BlockSpec((_OD, _BN), lambda j, c: (0, c * (_BATCH // _BN) + j)),
          pl.BlockSpec((_OD, _BN), lambda j, c: (0, c * (_BATCH // _BN) + j)),
          pl.BlockSpec((_OD, _BN), lambda j, c: (0, c * (_BATCH // _BN) + j)),
      ],
      out_specs=pl.BlockSpec((_OD, _BN), lambda j, c: (0, j)),
      out_shape=jax.ShapeDtypeStruct((_OD, _BATCH), jnp.float32),
      scratch_shapes=[
          pltpu.VMEM((_OD, _ED, _BN), jnp.float32),
          pltpu.VMEM((_OD, _BN), jnp.float32),
          pltpu.VMEM((_OD, _BN), jnp.float32),
      ],
  )(project_weight, eg_lo, eg_hi, lg)
  return out.T


# speedup vs baseline: 13.7145x; 1.6073x over previous
import functools

import jax
import jax.numpy as jnp
import numpy as np
from jax import lax
from jax.experimental import pallas as pl
from jax.experimental.pallas import tpu as pltpu
from jax.experimental.pallas import tpu_sc as plsc

_CAT_DIMS = [100000] * 26
_NF = 26
_BATCH = 4096
_ED = 16
_OD = 8
_ROWS = _BATCH * _NF
_CHUNK = 128
_NW = 32
_CPW = _ROWS // _CHUNK // _NW
_TCOL = _ROWS // _CHUNK

_OFFSETS = np.cumsum([0] + _CAT_DIMS[:-1]).astype(np.int32)

_CB = 65536
_K = _CB // 128
_NCB = -(-2600000 // _CB)
_FLAT = _NCB * _CB * 8


def _copy_body(in_ref, out_ref):
  x = in_ref[...]
  out_ref[...] = x.reshape(8, _K, 128).transpose(1, 0, 2).reshape(_K * 1024)


def _launder(tab, trow):
  return pl.pallas_call(
      _copy_body,
      grid=(_NCB,),
      in_specs=[pl.BlockSpec((8, _CB), lambda i, trow=trow: (trow, i))],
      out_specs=pl.BlockSpec((_K * 1024,), lambda i: (i,)),
      out_shape=jax.ShapeDtypeStruct((_FLAT,), jnp.float32),
  )(tab)


def _sc_gather(idx_hbm, tab, g4, idx_v, buf, sem):
  wid = lax.axis_index("s") * 2 + lax.axis_index("c")
  base = wid * _CPW

  def _fire(e):
    pltpu.sync_copy(idx_hbm.at[e].at[wid], idx_v.at[e])

    def body(j, carry):
      pltpu.async_copy(tab.at[idx_v.at[e].at[j]], buf.at[j].at[e], sem)
      return carry

    lax.fori_loop(0, _CPW, body, 0)

  for e in range(_OD):
    _fire(e)
  for e in range(_OD):
    pltpu.make_async_copy(idx_hbm.at[e].at[wid], idx_v.at[e], sem).wait()
  pltpu.sync_copy(buf, g4.at[pl.ds(base, _CPW)])


@functools.lru_cache(maxsize=1)
def _sc_gather_call():
  return pl.kernel(
      _sc_gather,
      out_type=jax.ShapeDtypeStruct((_TCOL, _OD, _CHUNK), jnp.float32),
      mesh=plsc.VectorSubcoreMesh(core_axis_name="c", subcore_axis_name="s"),
      scratch_types=[
          pltpu.VMEM((_OD, _CPW, _CHUNK), jnp.int32),
          pltpu.VMEM((_CPW, _OD, _CHUNK), jnp.float32),
          pltpu.SemaphoreType.DMA,
      ],
      compiler_params=pltpu.CompilerParams(use_tc_tiling_on_sc=False),
  )


_NBLK = 8
_BN = _BATCH // _NBLK


def _tc_body(w_ref, elo_ref, ehi_ref, l_ref, out_ref, sacc, qacc, lacc):
  c = pl.program_id(1)
  lo = elo_ref[...]
  hi = ehi_ref[...]
  lb = l_ref[...]
  ebsq = jnp.sum(lo * lo, axis=0) + jnp.sum(hi * hi, axis=0)

  @pl.when(c == 0)
  def _():
    for o in range(_OD):
      w = w_ref[o, 0]
      sacc[o, 0:8, :] = w * lo
      sacc[o, 8:16, :] = w * hi
      qacc[o, :] = (w * w) * ebsq
    lacc[...] = lb

  @pl.when(c != 0)
  def _():
    for o in range(_OD):
      w = w_ref[o, c]
      sacc[o, 0:8, :] = sacc[o, 0:8, :] + w * lo
      sacc[o, 8:16, :] = sacc[o, 8:16, :] + w * hi
      qacc[o, :] = qacc[o, :] + (w * w) * ebsq
    lacc[...] = lacc[...] + lb

  @pl.when(c == _NF - 1)
  def _():
    for o in range(_OD):
      s = sacc[o, :, :]
      out_ref[o, :] = lacc[o, :] + 0.5 * (jnp.sum(s * s, axis=0) - qacc[o, :])


def kernel(cat, lin_table, embed_table, project_weight):
  cat = jnp.asarray(cat, jnp.int32)
  idx = cat.T + jnp.asarray(_OFFSETS)[:, None]
  idx3 = idx.reshape(_NW, _CPW, _CHUNK)
  a0 = ((idx3 >> 7) << 10) + (idx3 & 127)
  planes = (jnp.arange(_OD, dtype=jnp.int32) << 7)
  idx_a = a0[None] + planes[:, None, None, None]

  et = embed_table.T
  lt = lin_table.T
  ltl = _launder(lt, 0)
  elo = _launder(et, 0)
  ehi = _launder(et, 1)

  gather = _sc_gather_call()
  lg4 = gather(idx_a, ltl)
  eg4_lo = gather(idx_a, elo)
  eg4_hi = gather(idx_a, ehi)
  lg = lg4.transpose(1, 0, 2).reshape(_OD, _ROWS)
  eg_lo = eg4_lo.transpose(1, 0, 2).reshape(_OD, _ROWS)
  eg_hi = eg4_hi.transpose(1, 0, 2).reshape(_OD, _ROWS)

  out = pl.pallas_call(
      _tc_body,
      grid=(_NBLK, _NF),
      in_specs=[
          pl.BlockSpec(memory_space=pltpu.SMEM),
          pl.BlockSpec((_OD, _BN), lambda j, c: (0, c * (_BATCH // _BN) + j)),
          pl.BlockSpec((_OD, _BN), lambda j, c: (0, c * (_BATCH // _BN) + j)),
          pl.BlockSpec((_OD, _BN), lambda j, c: (0, c * (_BATCH // _BN) + j)),
      ],
      out_specs=pl.BlockSpec((_OD, _BN), lambda j, c: (0, j)),
      out_shape=jax.ShapeDtypeStruct((_OD, _BATCH), jnp.float32),
      scratch_shapes=[
          pltpu.VMEM((_OD, _ED, _BN), jnp.float32),
          pltpu.VMEM((_OD, _BN), jnp.float32),
          pltpu.VMEM((_OD, _BN), jnp.float32),
      ],
  )(project_weight, eg_lo, eg_hi, lg)
  return out.T
